# baseline (speedup 1.0000x reference)
import functools

import jax
import jax.numpy as jnp
from jax import lax
from jax.experimental import pallas as pl
from jax.experimental.pallas import tpu as pltpu
from jax.experimental.pallas import tpu_sc as plsc

B = 16384
N = 1000000
MF_DIM = 16
MLP_HALF = 32
LANES = 128

MF_PACK = LANES // MF_DIM
MLP_PACK = LANES // MLP_HALF
MF_GRID = -(-(N // MF_PACK) // LANES)
MLP_GRID = -(-(N // MLP_PACK) // LANES)
MF_NP = MF_GRID * LANES
MLP_NP = MLP_GRID * LANES

_NC, _NS = 2, 16
_NW = _NC * _NS
_BPW = B // _NW
_CH = 128


def _make_pack(dim, pack, grid):
    def body(*refs):
        out_ref = refs[-1]
        for q in range(pack):
            out_ref[:, dim * q:dim * (q + 1)] = refs[q][...].T

    def apply(table):
        t_tr = table.T
        return pl.pallas_call(
            body,
            grid=(grid,),
            in_specs=[
                pl.BlockSpec((dim, LANES), lambda i, q=q: (0, i + grid * q))
                for q in range(pack)
            ],
            out_specs=pl.BlockSpec((LANES, LANES), lambda i: (i, 0)),
            out_shape=jax.ShapeDtypeStruct((grid * LANES, LANES), jnp.float32),
        )(*([t_tr] * pack))

    return apply


_pack_mf = _make_pack(MF_DIM, MF_PACK, MF_GRID)
_pack_mlp = _make_pack(MLP_HALF, MLP_PACK, MLP_GRID)


def _sc_gather_body(ju_hbm, ji_hbm, ju4_hbm, ji4_hbm,
                    mfu_p, mfi_p, mlpu_p, mlpi_p,
                    out_mfu, out_mfi, out_mlpu, out_mlpi,
                    ju_v, ji_v, ju4_v, ji4_v, bufs, sem):
    wid = lax.axis_index("s") * _NC + lax.axis_index("c")
    base = wid * _BPW
    pltpu.sync_copy(ju_hbm.at[pl.ds(base, _BPW)], ju_v)
    pltpu.sync_copy(ji_hbm.at[pl.ds(base, _BPW)], ji_v)
    pltpu.sync_copy(ju4_hbm.at[pl.ds(base, _BPW)], ju4_v)
    pltpu.sync_copy(ji4_hbm.at[pl.ds(base, _BPW)], ji4_v)

    def chunk(c, _):
        off = c * _CH
        cp = []
        for tab, idx_v, buf_i in ((mfu_p, ju_v, 0), (mfi_p, ji_v, 1),
                                  (mlpu_p, ju4_v, 2), (mlpi_p, ji4_v, 3)):
            cp.append(pltpu.async_copy(tab.at[idx_v.at[pl.ds(off, _CH)]],
                                       bufs.at[buf_i], sem))
        for x in cp:
            x.wait()
        for out, buf_i in ((out_mfu, 0), (out_mfi, 1),
                           (out_mlpu, 2), (out_mlpi, 3)):
            pltpu.sync_copy(bufs.at[buf_i], out.at[pl.ds(base + off, _CH)])
        return _

    lax.fori_loop(0, _BPW // _CH, chunk, 0)


@functools.cache
def _sc_gather():
    return pl.kernel(
        _sc_gather_body,
        out_type=[
            jax.ShapeDtypeStruct((B, LANES), jnp.float32),
            jax.ShapeDtypeStruct((B, LANES), jnp.float32),
            jax.ShapeDtypeStruct((B, LANES), jnp.float32),
            jax.ShapeDtypeStruct((B, LANES), jnp.float32),
        ],
        mesh=plsc.VectorSubcoreMesh(core_axis_name="c", subcore_axis_name="s"),
        scratch_types=[
            pltpu.VMEM((_BPW,), jnp.int32),
            pltpu.VMEM((_BPW,), jnp.int32),
            pltpu.VMEM((_BPW,), jnp.int32),
            pltpu.VMEM((_BPW,), jnp.int32),
            pltpu.VMEM((4, _CH, LANES), jnp.float32),
            pltpu.SemaphoreType.DMA,
        ],
    )


def _select_segment(packed, sel_ref, width):
    n = LANES // width
    sel = sel_ref[...]
    acc = None
    for q in range(n):
        part = jnp.where(sel == q, packed[:, q * width:(q + 1) * width], 0.0)
        acc = part if acc is None else acc + part
    return acc


def _tc_mlp_body(gmfu_ref, gmfi_ref, gmlpu_ref, gmlpi_ref,
                 su_ref, si_ref, tu_ref, ti_ref,
                 W1u_ref, W1i_ref, b1_ref, W2_ref, b2_ref,
                 Wfa_ref, Wfb_ref, bf_ref, out_ref):
    xmfu = _select_segment(gmfu_ref[...], su_ref, MF_DIM)
    xmfi = _select_segment(gmfi_ref[...], si_ref, MF_DIM)
    xmf = xmfu * xmfi
    xmlpu = _select_segment(gmlpu_ref[...], tu_ref, MLP_HALF)
    xmlpi = _select_segment(gmlpi_ref[...], ti_ref, MLP_HALF)
    h1 = xmlpu @ W1u_ref[...] + xmlpi @ W1i_ref[...] + b1_ref[...]
    h1 = jnp.maximum(h1, 0.0)
    h2 = jnp.maximum(h1 @ W2_ref[...] + b2_ref[...], 0.0)
    out_ref[...] = xmf @ Wfa_ref[...] + h2 @ Wfb_ref[...] + bf_ref[0, 0]


def _tc_mlp(gmfu, gmfi, gmlpu, gmlpi, su, si, tu, ti, W1, b1, W2, b2, Wf, bf):
    blk = 2048
    grid = (B // blk,)
    W1u = W1[:MLP_HALF]
    W1i = W1[MLP_HALF:]
    Wfa = Wf[:MF_DIM]
    Wfb = Wf[MF_DIM:]
    row = lambda i: (i, 0)
    rep = lambda i: (0, 0)
    return pl.pallas_call(
        _tc_mlp_body,
        grid=grid,
        in_specs=[
            pl.BlockSpec((blk, LANES), row),
            pl.BlockSpec((blk, LANES), row),
            pl.BlockSpec((blk, LANES), row),
            pl.BlockSpec((blk, LANES), row),
            pl.BlockSpec((blk, 1), row),
            pl.BlockSpec((blk, 1), row),
            pl.BlockSpec((blk, 1), row),
            pl.BlockSpec((blk, 1), row),
            pl.BlockSpec((MLP_HALF, 32), rep),
            pl.BlockSpec((MLP_HALF, 32), rep),
            pl.BlockSpec((1, 32), rep),
            pl.BlockSpec((32, 16), rep),
            pl.BlockSpec((1, 16), rep),
            pl.BlockSpec((MF_DIM, 1), rep),
            pl.BlockSpec((16, 1), rep),
            pl.BlockSpec((1, 1), rep),
        ],
        out_specs=pl.BlockSpec((blk, 1), row),
        out_shape=jax.ShapeDtypeStruct((B, 1), jnp.float32),
    )(gmfu, gmfi, gmlpu, gmlpi, su, si, tu, ti, W1u, W1i,
      b1.reshape(1, -1), W2, b2.reshape(1, -1), Wfa, Wfb, bf.reshape(1, 1))


def kernel(user, item, mf_user_embed, mf_item_embed, mlp_user_embed,
           mlp_item_embed, W1, b1, W2, b2, Wf, bf):
    user = user.astype(jnp.int32)
    item = item.astype(jnp.int32)
    mfu_p = _pack_mf(mf_user_embed)
    mfi_p = _pack_mf(mf_item_embed)
    mlpu_p = _pack_mlp(mlp_user_embed)
    mlpi_p = _pack_mlp(mlp_item_embed)
    gmfu, gmfi, gmlpu, gmlpi = _sc_gather()(
        user % MF_NP, item % MF_NP, user % MLP_NP, item % MLP_NP,
        mfu_p, mfi_p, mlpu_p, mlpi_p)
    su = (user // MF_NP).reshape(B, 1)
    si = (item // MF_NP).reshape(B, 1)
    tu = (user // MLP_NP).reshape(B, 1)
    ti = (item // MLP_NP).reshape(B, 1)
    return _tc_mlp(gmfu, gmfi, gmlpu, gmlpi, su, si, tu, ti,
                   W1, b1, W2, b2, Wf, bf)


# speedup vs baseline: 5.9554x; 5.9554x over previous
import functools

import jax
import jax.numpy as jnp
from jax import lax
from jax.experimental import pallas as pl
from jax.experimental.pallas import tpu as pltpu
from jax.experimental.pallas import tpu_sc as plsc

B = 16384
N = 1000000
MF_DIM = 16
MLP_HALF = 32
LANES = 128

MF_PACK = LANES // MF_DIM
MLP_PACK = LANES // MLP_HALF
PAD_N = -(-N // LANES) * LANES
PACK_J = 1664
assert PAD_N % PACK_J == 0
MF_GRID = -(-(N // MF_PACK) // PACK_J)
MLP_GRID = -(-(N // MLP_PACK) // PACK_J)
MF_NP = MF_GRID * PACK_J
MLP_NP = MLP_GRID * PACK_J
MF_LAST = PAD_N - MF_NP
MLP_LAST = PAD_N - MLP_NP
assert (MF_PACK - 1) * MF_NP >= MF_LAST
assert (MLP_PACK - 1) * MLP_NP >= MLP_LAST

_NC, _NS = 2, 16
_NW = _NC * _NS
_BPW = B // _NW
_CH = 128


def _make_pack(dim, pack, grid, last_start):
    starts = [q * grid for q in range(pack - 1)] + [last_start // PACK_J]

    def body(*refs):
        out_ref = refs[-1]
        stacked = jnp.concatenate([refs[q][...] for q in range(pack)], axis=0)
        out_ref[...] = stacked.T

    def apply(table):
        t_tr = table.T
        return pl.pallas_call(
            body,
            grid=(grid,),
            in_specs=[
                pl.BlockSpec((dim, PACK_J), lambda i, s=s: (0, i + s))
                for s in starts
            ],
            out_specs=pl.BlockSpec((PACK_J, LANES), lambda i: (i, 0)),
            out_shape=jax.ShapeDtypeStruct((grid * PACK_J, LANES),
                                           jnp.float32),
        )(*([t_tr] * pack))

    return apply


_pack_mf = _make_pack(MF_DIM, MF_PACK, MF_GRID, MF_LAST)
_pack_mlp = _make_pack(MLP_HALF, MLP_PACK, MLP_GRID, MLP_LAST)


def _sc_gather_body(ju_hbm, ji_hbm, ju4_hbm, ji4_hbm,
                    mfu_p, mfi_p, mlpu_p, mlpi_p,
                    out_mfu, out_mfi, out_mlpu, out_mlpi,
                    ju_v, ji_v, ju4_v, ji4_v, bufs, sem):
    wid = lax.axis_index("s") * _NC + lax.axis_index("c")
    base = wid * _BPW
    pltpu.sync_copy(ju_hbm.at[pl.ds(base, _BPW)], ju_v)
    pltpu.sync_copy(ji_hbm.at[pl.ds(base, _BPW)], ji_v)
    pltpu.sync_copy(ju4_hbm.at[pl.ds(base, _BPW)], ju4_v)
    pltpu.sync_copy(ji4_hbm.at[pl.ds(base, _BPW)], ji4_v)

    def chunk(c, _):
        off = c * _CH
        cp = []
        for tab, idx_v, buf_i in ((mfu_p, ju_v, 0), (mfi_p, ji_v, 1),
                                  (mlpu_p, ju4_v, 2), (mlpi_p, ji4_v, 3)):
            cp.append(pltpu.async_copy(tab.at[idx_v.at[pl.ds(off, _CH)]],
                                       bufs.at[buf_i], sem))
        for x in cp:
            x.wait()
        for out, buf_i in ((out_mfu, 0), (out_mfi, 1),
                           (out_mlpu, 2), (out_mlpi, 3)):
            pltpu.sync_copy(bufs.at[buf_i], out.at[pl.ds(base + off, _CH)])
        return _

    lax.fori_loop(0, _BPW // _CH, chunk, 0)


@functools.cache
def _sc_gather():
    return pl.kernel(
        _sc_gather_body,
        out_type=[
            jax.ShapeDtypeStruct((B, LANES), jnp.float32),
            jax.ShapeDtypeStruct((B, LANES), jnp.float32),
            jax.ShapeDtypeStruct((B, LANES), jnp.float32),
            jax.ShapeDtypeStruct((B, LANES), jnp.float32),
        ],
        mesh=plsc.VectorSubcoreMesh(core_axis_name="c", subcore_axis_name="s"),
        scratch_types=[
            pltpu.VMEM((_BPW,), jnp.int32),
            pltpu.VMEM((_BPW,), jnp.int32),
            pltpu.VMEM((_BPW,), jnp.int32),
            pltpu.VMEM((_BPW,), jnp.int32),
            pltpu.VMEM((4, _CH, LANES), jnp.float32),
            pltpu.SemaphoreType.DMA,
        ],
    )


def _select_segment(packed, sel_ref, width):
    n = LANES // width
    sel = sel_ref[...]
    acc = None
    for q in range(n):
        part = jnp.where(sel == q, packed[:, q * width:(q + 1) * width], 0.0)
        acc = part if acc is None else acc + part
    return acc


def _tc_mlp_body(gmfu_ref, gmfi_ref, gmlpu_ref, gmlpi_ref,
                 su_ref, si_ref, tu_ref, ti_ref,
                 W1u_ref, W1i_ref, b1_ref, W2_ref, b2_ref,
                 Wfa_ref, Wfb_ref, bf_ref, out_ref):
    xmfu = _select_segment(gmfu_ref[...], su_ref, MF_DIM)
    xmfi = _select_segment(gmfi_ref[...], si_ref, MF_DIM)
    xmf = xmfu * xmfi
    xmlpu = _select_segment(gmlpu_ref[...], tu_ref, MLP_HALF)
    xmlpi = _select_segment(gmlpi_ref[...], ti_ref, MLP_HALF)
    h1 = xmlpu @ W1u_ref[...] + xmlpi @ W1i_ref[...] + b1_ref[...]
    h1 = jnp.maximum(h1, 0.0)
    h2 = jnp.maximum(h1 @ W2_ref[...] + b2_ref[...], 0.0)
    out_ref[...] = xmf @ Wfa_ref[...] + h2 @ Wfb_ref[...] + bf_ref[0, 0]


def _tc_mlp(gmfu, gmfi, gmlpu, gmlpi, su, si, tu, ti, W1, b1, W2, b2, Wf, bf):
    blk = 2048
    grid = (B // blk,)
    W1u = W1[:MLP_HALF]
    W1i = W1[MLP_HALF:]
    Wfa = Wf[:MF_DIM]
    Wfb = Wf[MF_DIM:]
    row = lambda i: (i, 0)
    rep = lambda i: (0, 0)
    return pl.pallas_call(
        _tc_mlp_body,
        grid=grid,
        in_specs=[
            pl.BlockSpec((blk, LANES), row),
            pl.BlockSpec((blk, LANES), row),
            pl.BlockSpec((blk, LANES), row),
            pl.BlockSpec((blk, LANES), row),
            pl.BlockSpec((blk, 1), row),
            pl.BlockSpec((blk, 1), row),
            pl.BlockSpec((blk, 1), row),
            pl.BlockSpec((blk, 1), row),
            pl.BlockSpec((MLP_HALF, 32), rep),
            pl.BlockSpec((MLP_HALF, 32), rep),
            pl.BlockSpec((1, 32), rep),
            pl.BlockSpec((32, 16), rep),
            pl.BlockSpec((1, 16), rep),
            pl.BlockSpec((MF_DIM, 1), rep),
            pl.BlockSpec((16, 1), rep),
            pl.BlockSpec((1, 1), rep),
        ],
        out_specs=pl.BlockSpec((blk, 1), row),
        out_shape=jax.ShapeDtypeStruct((B, 1), jnp.float32),
    )(gmfu, gmfi, gmlpu, gmlpi, su, si, tu, ti, W1u, W1i,
      b1.reshape(1, -1), W2, b2.reshape(1, -1), Wfa, Wfb, bf.reshape(1, 1))


def kernel(user, item, mf_user_embed, mf_item_embed, mlp_user_embed,
           mlp_item_embed, W1, b1, W2, b2, Wf, bf):
    user = user.astype(jnp.int32)
    item = item.astype(jnp.int32)
    mfu_p = _pack_mf(mf_user_embed)
    mfi_p = _pack_mf(mf_item_embed)
    mlpu_p = _pack_mlp(mlp_user_embed)
    mlpi_p = _pack_mlp(mlp_item_embed)
    def phase_split(r, np_, pack, last_start):
        q = r // np_
        j = r - jnp.where(q == pack - 1, last_start, q * np_)
        return q, j

    su_f, ju = phase_split(user, MF_NP, MF_PACK, MF_LAST)
    si_f, ji = phase_split(item, MF_NP, MF_PACK, MF_LAST)
    tu_f, ju4 = phase_split(user, MLP_NP, MLP_PACK, MLP_LAST)
    ti_f, ji4 = phase_split(item, MLP_NP, MLP_PACK, MLP_LAST)
    gmfu, gmfi, gmlpu, gmlpi = _sc_gather()(
        ju, ji, ju4, ji4, mfu_p, mfi_p, mlpu_p, mlpi_p)
    su = su_f.reshape(B, 1)
    si = si_f.reshape(B, 1)
    tu = tu_f.reshape(B, 1)
    ti = ti_f.reshape(B, 1)
    return _tc_mlp(gmfu, gmfi, gmlpu, gmlpi, su, si, tu, ti,
                   W1, b1, W2, b2, Wf, bf)


# speedup vs baseline: 6.9132x; 1.1608x over previous
import functools

import jax
import jax.numpy as jnp
from jax import lax
from jax.experimental import pallas as pl
from jax.experimental.pallas import tpu as pltpu
from jax.experimental.pallas import tpu_sc as plsc

B = 16384
N = 1000000
MF_DIM = 16
MLP_HALF = 32
LANES = 128

MF_PACK = LANES // MF_DIM
MLP_PACK = LANES // MLP_HALF
PAD_N = -(-N // LANES) * LANES
PACK_J = 1664
assert PAD_N % PACK_J == 0
MF_GRID = -(-(N // MF_PACK) // PACK_J)
MLP_GRID = -(-(N // MLP_PACK) // PACK_J)
MF_NP = MF_GRID * PACK_J
MLP_NP = MLP_GRID * PACK_J
MF_LAST = PAD_N - MF_NP
MLP_LAST = PAD_N - MLP_NP
assert (MF_PACK - 1) * MF_NP >= MF_LAST
assert (MLP_PACK - 1) * MLP_NP >= MLP_LAST

_NC, _NS = 2, 16
_NW = _NC * _NS
_BPW = B // _NW
_CH = 128


def _make_pack(dim, pack, grid, last_start):
    starts = [q * grid for q in range(pack - 1)] + [last_start // PACK_J]

    def body(*refs):
        out_ref = refs[-1]
        stacked = jnp.concatenate([refs[q][...] for q in range(pack)], axis=0)
        out_ref[...] = stacked.T

    def apply(table):
        t_tr = table.T
        return pl.pallas_call(
            body,
            grid=(grid,),
            in_specs=[
                pl.BlockSpec((dim, PACK_J), lambda i, s=s: (0, i + s))
                for s in starts
            ],
            out_specs=pl.BlockSpec((PACK_J, LANES), lambda i: (i, 0)),
            out_shape=jax.ShapeDtypeStruct((grid * PACK_J, LANES),
                                           jnp.float32),
        )(*([t_tr] * pack))

    return apply


_pack_mf = _make_pack(MF_DIM, MF_PACK, MF_GRID, MF_LAST)
_pack_mlp = _make_pack(MLP_HALF, MLP_PACK, MLP_GRID, MLP_LAST)


def _sc_gather_body(ju_hbm, ji_hbm, ju4_hbm, ji4_hbm,
                    mfu_p, mfi_p, mlpu_p, mlpi_p,
                    out_mfu, out_mfi, out_mlpu, out_mlpi,
                    ju_v, ji_v, ju4_v, ji4_v, bufs, sem):
    wid = lax.axis_index("s") * _NC + lax.axis_index("c")
    base = wid * _BPW
    pltpu.sync_copy(ju_hbm.at[pl.ds(base, _BPW)], ju_v)
    pltpu.sync_copy(ji_hbm.at[pl.ds(base, _BPW)], ji_v)
    pltpu.sync_copy(ju4_hbm.at[pl.ds(base, _BPW)], ju4_v)
    pltpu.sync_copy(ji4_hbm.at[pl.ds(base, _BPW)], ji4_v)

    def chunk(c, _):
        off = c * _CH
        cp = []
        for tab, idx_v, buf_i in ((mfu_p, ju_v, 0), (mfi_p, ji_v, 1),
                                  (mlpu_p, ju4_v, 2), (mlpi_p, ji4_v, 3)):
            cp.append(pltpu.async_copy(tab.at[idx_v.at[pl.ds(off, _CH)]],
                                       bufs.at[buf_i], sem))
        for x in cp:
            x.wait()
        for out, buf_i in ((out_mfu, 0), (out_mfi, 1),
                           (out_mlpu, 2), (out_mlpi, 3)):
            pltpu.sync_copy(bufs.at[buf_i], out.at[pl.ds(base + off, _CH)])
        return _

    lax.fori_loop(0, _BPW // _CH, chunk, 0)


@functools.cache
def _sc_gather():
    return pl.kernel(
        _sc_gather_body,
        out_type=[
            jax.ShapeDtypeStruct((B, LANES), jnp.float32),
            jax.ShapeDtypeStruct((B, LANES), jnp.float32),
            jax.ShapeDtypeStruct((B, LANES), jnp.float32),
            jax.ShapeDtypeStruct((B, LANES), jnp.float32),
        ],
        mesh=plsc.VectorSubcoreMesh(core_axis_name="c", subcore_axis_name="s"),
        scratch_types=[
            pltpu.VMEM((_BPW,), jnp.int32),
            pltpu.VMEM((_BPW,), jnp.int32),
            pltpu.VMEM((_BPW,), jnp.int32),
            pltpu.VMEM((_BPW,), jnp.int32),
            pltpu.VMEM((4, _CH, LANES), jnp.float32),
            pltpu.SemaphoreType.DMA,
        ],
    )


def _mask(sel_ref, width, blk):
    lane_q = lax.broadcasted_iota(jnp.int32, (blk, LANES), 1) // width
    return lane_q == sel_ref[...]


def _tc_mlp_body(gmfu_ref, gmfi_ref, gmlpu_ref, gmlpi_ref,
                 su_ref, si_ref, tu_ref, ti_ref,
                 F16_ref, W1u128_ref, W1i128_ref, b1_ref, W2_ref, b2_ref,
                 Wfa_ref, Wfb_ref, bf_ref, out_ref):
    blk = gmfu_ref.shape[0]
    zu = jnp.where(_mask(su_ref, MF_DIM, blk), gmfu_ref[...], 0.0)
    zi = jnp.where(_mask(si_ref, MF_DIM, blk), gmfi_ref[...], 0.0)
    xmf = (zu @ F16_ref[...]) * (zi @ F16_ref[...])
    yu = jnp.where(_mask(tu_ref, MLP_HALF, blk), gmlpu_ref[...], 0.0)
    yi = jnp.where(_mask(ti_ref, MLP_HALF, blk), gmlpi_ref[...], 0.0)
    h1 = yu @ W1u128_ref[...] + yi @ W1i128_ref[...] + b1_ref[...]
    h1 = jnp.maximum(h1, 0.0)
    h2 = jnp.maximum(h1 @ W2_ref[...] + b2_ref[...], 0.0)
    out_ref[...] = xmf @ Wfa_ref[...] + h2 @ Wfb_ref[...] + bf_ref[0, 0]


def _tc_mlp(gmfu, gmfi, gmlpu, gmlpi, su, si, tu, ti, W1, b1, W2, b2, Wf, bf):
    blk = 2048
    grid = (B // blk,)
    F16 = (lax.broadcasted_iota(jnp.int32, (LANES, MF_DIM), 0) % MF_DIM
           == lax.broadcasted_iota(jnp.int32, (LANES, MF_DIM), 1)
           ).astype(jnp.float32)
    F32 = (lax.broadcasted_iota(jnp.int32, (LANES, MLP_HALF), 0) % MLP_HALF
           == lax.broadcasted_iota(jnp.int32, (LANES, MLP_HALF), 1)
           ).astype(jnp.float32)
    W1u128 = F32 @ W1[:MLP_HALF]
    W1i128 = F32 @ W1[MLP_HALF:]
    Wfa = Wf[:MF_DIM]
    Wfb = Wf[MF_DIM:]
    row = lambda i: (i, 0)
    rep = lambda i: (0, 0)
    return pl.pallas_call(
        _tc_mlp_body,
        grid=grid,
        in_specs=[
            pl.BlockSpec((blk, LANES), row),
            pl.BlockSpec((blk, LANES), row),
            pl.BlockSpec((blk, LANES), row),
            pl.BlockSpec((blk, LANES), row),
            pl.BlockSpec((blk, 1), row),
            pl.BlockSpec((blk, 1), row),
            pl.BlockSpec((blk, 1), row),
            pl.BlockSpec((blk, 1), row),
            pl.BlockSpec((LANES, MF_DIM), rep),
            pl.BlockSpec((LANES, 32), rep),
            pl.BlockSpec((LANES, 32), rep),
            pl.BlockSpec((1, 32), rep),
            pl.BlockSpec((32, 16), rep),
            pl.BlockSpec((1, 16), rep),
            pl.BlockSpec((MF_DIM, 1), rep),
            pl.BlockSpec((16, 1), rep),
            pl.BlockSpec((1, 1), rep),
        ],
        out_specs=pl.BlockSpec((blk, 1), row),
        out_shape=jax.ShapeDtypeStruct((B, 1), jnp.float32),
    )(gmfu, gmfi, gmlpu, gmlpi, su, si, tu, ti, F16, W1u128, W1i128,
      b1.reshape(1, -1), W2, b2.reshape(1, -1), Wfa, Wfb, bf.reshape(1, 1))


def kernel(user, item, mf_user_embed, mf_item_embed, mlp_user_embed,
           mlp_item_embed, W1, b1, W2, b2, Wf, bf):
    user = user.astype(jnp.int32)
    item = item.astype(jnp.int32)
    mfu_p = _pack_mf(mf_user_embed)
    mfi_p = _pack_mf(mf_item_embed)
    mlpu_p = _pack_mlp(mlp_user_embed)
    mlpi_p = _pack_mlp(mlp_item_embed)
    def phase_split(r, np_, pack, last_start):
        q = r // np_
        j = r - jnp.where(q == pack - 1, last_start, q * np_)
        return q, j

    su_f, ju = phase_split(user, MF_NP, MF_PACK, MF_LAST)
    si_f, ji = phase_split(item, MF_NP, MF_PACK, MF_LAST)
    tu_f, ju4 = phase_split(user, MLP_NP, MLP_PACK, MLP_LAST)
    ti_f, ji4 = phase_split(item, MLP_NP, MLP_PACK, MLP_LAST)
    gmfu, gmfi, gmlpu, gmlpi = _sc_gather()(
        ju, ji, ju4, ji4, mfu_p, mfi_p, mlpu_p, mlpi_p)
    su = su_f.reshape(B, 1)
    si = si_f.reshape(B, 1)
    tu = tu_f.reshape(B, 1)
    ti = ti_f.reshape(B, 1)
    return _tc_mlp(gmfu, gmfi, gmlpu, gmlpi, su, si, tu, ti,
                   W1, b1, W2, b2, Wf, bf)


# speedup vs baseline: 6.9230x; 1.0014x over previous
import functools

import jax
import jax.numpy as jnp
from jax import lax
from jax.experimental import pallas as pl
from jax.experimental.pallas import tpu as pltpu
from jax.experimental.pallas import tpu_sc as plsc

B = 16384
N = 1000000
MF_DIM = 16
MLP_HALF = 32
LANES = 128

MF_PACK = LANES // MF_DIM
MLP_PACK = LANES // MLP_HALF
PAD_N = -(-N // LANES) * LANES
PACK_J = 1664
assert PAD_N % PACK_J == 0
MF_GRID = -(-(N // MF_PACK) // PACK_J)
MLP_GRID = -(-(N // MLP_PACK) // PACK_J)
MF_NP = MF_GRID * PACK_J
MLP_NP = MLP_GRID * PACK_J
MF_LAST = PAD_N - MF_NP
MLP_LAST = PAD_N - MLP_NP
assert (MF_PACK - 1) * MF_NP >= MF_LAST
assert (MLP_PACK - 1) * MLP_NP >= MLP_LAST

_NC, _NS = 2, 16
_NW = _NC * _NS
_BPW = B // _NW
_CH = 256


def _make_pack(dim, pack, grid, last_start):
    starts = [q * grid for q in range(pack - 1)] + [last_start // PACK_J]

    def body(*refs):
        out_ref = refs[-1]
        stacked = jnp.concatenate([refs[q][...] for q in range(pack)], axis=0)
        out_ref[...] = stacked.T

    def apply(table):
        t_tr = table.T
        return pl.pallas_call(
            body,
            grid=(grid,),
            in_specs=[
                pl.BlockSpec((dim, PACK_J), lambda i, s=s: (0, i + s))
                for s in starts
            ],
            out_specs=pl.BlockSpec((PACK_J, LANES), lambda i: (i, 0)),
            out_shape=jax.ShapeDtypeStruct((grid * PACK_J, LANES),
                                           jnp.float32),
        )(*([t_tr] * pack))

    return apply


_pack_mf = _make_pack(MF_DIM, MF_PACK, MF_GRID, MF_LAST)
_pack_mlp = _make_pack(MLP_HALF, MLP_PACK, MLP_GRID, MLP_LAST)


def _sc_gather_body(ju_hbm, ji_hbm, tab_u, tab_i, out_u, out_i,
                    ju_v, ji_v, bufs, sem):
    wid = lax.axis_index("s") * _NC + lax.axis_index("c")
    base = wid * _BPW
    pltpu.sync_copy(ju_hbm.at[pl.ds(base, _BPW)], ju_v)
    pltpu.sync_copy(ji_hbm.at[pl.ds(base, _BPW)], ji_v)

    def chunk(c, _):
        off = c * _CH
        cp1 = pltpu.async_copy(tab_u.at[ju_v.at[pl.ds(off, _CH)]],
                               bufs.at[0], sem)
        cp2 = pltpu.async_copy(tab_i.at[ji_v.at[pl.ds(off, _CH)]],
                               bufs.at[1], sem)
        cp1.wait()
        cp2.wait()
        pltpu.sync_copy(bufs.at[0], out_u.at[pl.ds(base + off, _CH)])
        pltpu.sync_copy(bufs.at[1], out_i.at[pl.ds(base + off, _CH)])
        return _

    lax.fori_loop(0, _BPW // _CH, chunk, 0)


@functools.cache
def _sc_gather():
    return pl.kernel(
        _sc_gather_body,
        out_type=[
            jax.ShapeDtypeStruct((B, LANES), jnp.float32),
            jax.ShapeDtypeStruct((B, LANES), jnp.float32),
        ],
        mesh=plsc.VectorSubcoreMesh(core_axis_name="c", subcore_axis_name="s"),
        scratch_types=[
            pltpu.VMEM((_BPW,), jnp.int32),
            pltpu.VMEM((_BPW,), jnp.int32),
            pltpu.VMEM((2, _CH, LANES), jnp.float32),
            pltpu.SemaphoreType.DMA,
        ],
    )


def _mask(sel_ref, width, blk):
    lane_q = lax.broadcasted_iota(jnp.int32, (blk, LANES), 1) // width
    return lane_q == sel_ref[...]


def _tc_mlp_body(gmfu_ref, gmfi_ref, gmlpu_ref, gmlpi_ref,
                 su_ref, si_ref, tu_ref, ti_ref,
                 F16_ref, W1u128_ref, W1i128_ref, b1_ref, W2_ref, b2_ref,
                 Wfa_ref, Wfb_ref, bf_ref, out_ref):
    blk = gmfu_ref.shape[0]
    zu = jnp.where(_mask(su_ref, MF_DIM, blk), gmfu_ref[...], 0.0)
    zi = jnp.where(_mask(si_ref, MF_DIM, blk), gmfi_ref[...], 0.0)
    xmf = (zu @ F16_ref[...]) * (zi @ F16_ref[...])
    yu = jnp.where(_mask(tu_ref, MLP_HALF, blk), gmlpu_ref[...], 0.0)
    yi = jnp.where(_mask(ti_ref, MLP_HALF, blk), gmlpi_ref[...], 0.0)
    h1 = yu @ W1u128_ref[...] + yi @ W1i128_ref[...] + b1_ref[...]
    h1 = jnp.maximum(h1, 0.0)
    h2 = jnp.maximum(h1 @ W2_ref[...] + b2_ref[...], 0.0)
    out_ref[...] = xmf @ Wfa_ref[...] + h2 @ Wfb_ref[...] + bf_ref[0, 0]


def _tc_mlp(gmfu, gmfi, gmlpu, gmlpi, su, si, tu, ti, W1, b1, W2, b2, Wf, bf):
    blk = 2048
    grid = (B // blk,)
    F16 = (lax.broadcasted_iota(jnp.int32, (LANES, MF_DIM), 0) % MF_DIM
           == lax.broadcasted_iota(jnp.int32, (LANES, MF_DIM), 1)
           ).astype(jnp.float32)
    F32 = (lax.broadcasted_iota(jnp.int32, (LANES, MLP_HALF), 0) % MLP_HALF
           == lax.broadcasted_iota(jnp.int32, (LANES, MLP_HALF), 1)
           ).astype(jnp.float32)
    W1u128 = F32 @ W1[:MLP_HALF]
    W1i128 = F32 @ W1[MLP_HALF:]
    Wfa = Wf[:MF_DIM]
    Wfb = Wf[MF_DIM:]
    row = lambda i: (i, 0)
    rep = lambda i: (0, 0)
    return pl.pallas_call(
        _tc_mlp_body,
        grid=grid,
        in_specs=[
            pl.BlockSpec((blk, LANES), row),
            pl.BlockSpec((blk, LANES), row),
            pl.BlockSpec((blk, LANES), row),
            pl.BlockSpec((blk, LANES), row),
            pl.BlockSpec((blk, 1), row),
            pl.BlockSpec((blk, 1), row),
            pl.BlockSpec((blk, 1), row),
            pl.BlockSpec((blk, 1), row),
            pl.BlockSpec((LANES, MF_DIM), rep),
            pl.BlockSpec((LANES, 32), rep),
            pl.BlockSpec((LANES, 32), rep),
            pl.BlockSpec((1, 32), rep),
            pl.BlockSpec((32, 16), rep),
            pl.BlockSpec((1, 16), rep),
            pl.BlockSpec((MF_DIM, 1), rep),
            pl.BlockSpec((16, 1), rep),
            pl.BlockSpec((1, 1), rep),
        ],
        out_specs=pl.BlockSpec((blk, 1), row),
        out_shape=jax.ShapeDtypeStruct((B, 1), jnp.float32),
    )(gmfu, gmfi, gmlpu, gmlpi, su, si, tu, ti, F16, W1u128, W1i128,
      b1.reshape(1, -1), W2, b2.reshape(1, -1), Wfa, Wfb, bf.reshape(1, 1))


def kernel(user, item, mf_user_embed, mf_item_embed, mlp_user_embed,
           mlp_item_embed, W1, b1, W2, b2, Wf, bf):
    user = user.astype(jnp.int32)
    item = item.astype(jnp.int32)
    mfu_p = _pack_mf(mf_user_embed)
    mfi_p = _pack_mf(mf_item_embed)
    mlpu_p = _pack_mlp(mlp_user_embed)
    mlpi_p = _pack_mlp(mlp_item_embed)
    def phase_split(r, np_, pack, last_start):
        q = r // np_
        j = r - jnp.where(q == pack - 1, last_start, q * np_)
        return q, j

    su_f, ju = phase_split(user, MF_NP, MF_PACK, MF_LAST)
    si_f, ji = phase_split(item, MF_NP, MF_PACK, MF_LAST)
    tu_f, ju4 = phase_split(user, MLP_NP, MLP_PACK, MLP_LAST)
    ti_f, ji4 = phase_split(item, MLP_NP, MLP_PACK, MLP_LAST)
    gmfu, gmfi = _sc_gather()(ju, ji, mfu_p, mfi_p)
    gmlpu, gmlpi = _sc_gather()(ju4, ji4, mlpu_p, mlpi_p)
    su = su_f.reshape(B, 1)
    si = si_f.reshape(B, 1)
    tu = tu_f.reshape(B, 1)
    ti = ti_f.reshape(B, 1)
    return _tc_mlp(gmfu, gmfi, gmlpu, gmlpi, su, si, tu, ti,
                   W1, b1, W2, b2, Wf, bf)
